# Initial kernel scaffold; baseline (speedup 1.0000x reference)
import jax
import jax.numpy as jnp
from jax.experimental import pallas as pl


def kernel(x, edge_index, target_indices, W0_self, W0_neigh, b0, W1_self, W1_neigh, b1):
    raise NotImplementedError("write your pallas kernel here")


# speedup vs baseline: 2.9531x; 2.9531x over previous
import functools

import jax
import jax.numpy as jnp
from jax import lax
from jax.experimental import pallas as pl
from jax.experimental.pallas import tpu as pltpu
from jax.experimental.pallas import tpu_sc as plsc

N_NODES = 10000
NPAD = 10240
D = 256
H = 128
E = 160000
CHUNK = 64
GC = 8
EPAD = 163840
CHUNKS_PER_TILE = EPAD // 16 // CHUNK
GROUPS_PER_TILE = CHUNKS_PER_TILE // GC
EDGES_PER_GROUP = GC * CHUNK
ROWS_PER_TILE = NPAD // 16
NT = 1024
TPT = NT // 16

_mesh = plsc.VectorSubcoreMesh(core_axis_name="c", subcore_axis_name="s")
_sc_params = pltpu.CompilerParams(needs_layout_passes=False)


def _edge_sweep(x_hbm, src_hbm, dst_hbm, s, srcg, dstg, rows0, rows1, acc,
                gsem0, gsem1, degloc=None):
    ones16 = jnp.ones((16,), jnp.float32)
    rows = (rows0, rows1)
    gsem = (gsem0, gsem1)
    base = s * GROUPS_PER_TILE

    def group(g, carry):
        pltpu.sync_copy(src_hbm.at[pl.ds((base + g) * GC, GC)], srcg)
        pltpu.sync_copy(dst_hbm.at[pl.ds((base + g) * GC, GC)], dstg)
        pltpu.async_copy(x_hbm.at[srcg.at[0]], rows[0], gsem[0])
        for b in range(GC):
            p = b % 2
            if b + 1 < GC:
                pltpu.async_copy(x_hbm.at[srcg.at[b + 1]], rows[1 - p],
                                 gsem[1 - p])
            if degloc is not None:
                for k in range(CHUNK // 16):
                    idx16 = dstg[b, pl.ds(k * 16, 16)]
                    plsc.addupdate_scatter(degloc, [idx16], ones16)
            pltpu.make_async_copy(x_hbm.at[srcg.at[b]], rows[p], gsem[p]).wait()
            pltpu.sync_copy(rows[p], acc.at[dstg.at[b]], add=True)
        return carry

    lax.fori_loop(0, GROUPS_PER_TILE, group, 0)


def _segsum0_body(xa_hbm, xb_hbm, src_hbm, dst_hbm, zrows_hbm, zvec_hbm,
                  agg_hbm, degp_hbm, srcg, dstg, rows0, rows1, degloc, acc,
                  gsem0, gsem1):
    c = lax.axis_index("c")
    s = lax.axis_index("s")
    pltpu.sync_copy(zrows_hbm, acc.at[pl.ds(s * ROWS_PER_TILE, ROWS_PER_TILE)])
    pltpu.sync_copy(zvec_hbm, degloc)
    plsc.subcore_barrier()

    pl.when(c == 0)(lambda: _edge_sweep(
        xa_hbm, src_hbm, dst_hbm, s, srcg, dstg, rows0, rows1, acc,
        gsem0, gsem1, degloc))
    pl.when(c == 1)(lambda: _edge_sweep(
        xb_hbm, src_hbm, dst_hbm, s, srcg, dstg, rows0, rows1, acc,
        gsem0, gsem1, degloc))
    pltpu.sync_copy(degloc, degp_hbm.at[c, s])
    plsc.subcore_barrier()
    pltpu.sync_copy(acc.at[pl.ds(s * ROWS_PER_TILE, ROWS_PER_TILE)],
                    agg_hbm.at[c, pl.ds(s * ROWS_PER_TILE, ROWS_PER_TILE)])


_seg0 = functools.partial(
    pl.kernel,
    mesh=_mesh,
    out_type=[
        jax.ShapeDtypeStruct((2, NPAD, H), jnp.float32),
        jax.ShapeDtypeStruct((2, 16, NPAD), jnp.float32),
    ],
    scratch_types=[
        pltpu.VMEM((GC, CHUNK), jnp.int32),
        pltpu.VMEM((GC, CHUNK), jnp.int32),
        pltpu.VMEM((CHUNK, H), jnp.float32),
        pltpu.VMEM((CHUNK, H), jnp.float32),
        pltpu.VMEM((NPAD,), jnp.float32),
        pltpu.VMEM_SHARED((NPAD, H), jnp.float32),
        pltpu.SemaphoreType.DMA,
        pltpu.SemaphoreType.DMA,
    ],
    compiler_params=_sc_params,
)(_segsum0_body)


def _segsum1_body(ha_hbm, hb_hbm, src_hbm, dst_hbm, ti_hbm, dinv_hbm,
                  zrows_hbm, aggt_hbm, ht_hbm,
                  srcg, dstg, tiv, rows0, rows1, dinvv, acc, gsem0, gsem1):
    c = lax.axis_index("c")
    s = lax.axis_index("s")
    pltpu.sync_copy(zrows_hbm, acc.at[pl.ds(s * ROWS_PER_TILE, ROWS_PER_TILE)])
    pltpu.sync_copy(ti_hbm.at[pl.ds(s * TPT, TPT)], tiv)
    pltpu.sync_copy(dinv_hbm, dinvv)
    plsc.subcore_barrier()

    def run(h_hbm):
        _edge_sweep(h_hbm, src_hbm, dst_hbm, s, srcg, dstg, rows0, rows1,
                    acc, gsem0, gsem1)
        pltpu.async_copy(h_hbm.at[tiv], rows1, gsem1).wait()

    pl.when(c == 0)(lambda: run(ha_hbm))
    pl.when(c == 1)(lambda: run(hb_hbm))
    plsc.subcore_barrier()
    pltpu.async_copy(acc.at[tiv], rows0, gsem0).wait()

    for g in range(TPT // 16):
        tiv16 = tiv[pl.ds(g * 16, 16)]
        dinv16 = plsc.load_gather(dinvv, [tiv16])
        for l in range(16):
            d = dinv16[l]
            r = g * 16 + l
            for k in range(H // 16):
                rows0[r, pl.ds(k * 16, 16)] = rows0[r, pl.ds(k * 16, 16)] * d

    pltpu.sync_copy(rows0, aggt_hbm.at[c, pl.ds(s * TPT, TPT)])
    pltpu.sync_copy(rows1, ht_hbm.at[c, pl.ds(s * TPT, TPT)])


_seg1 = functools.partial(
    pl.kernel,
    mesh=_mesh,
    out_type=[
        jax.ShapeDtypeStruct((2, NT, H), jnp.float32),
        jax.ShapeDtypeStruct((2, NT, H), jnp.float32),
    ],
    scratch_types=[
        pltpu.VMEM((GC, CHUNK), jnp.int32),
        pltpu.VMEM((GC, CHUNK), jnp.int32),
        pltpu.VMEM((TPT,), jnp.int32),
        pltpu.VMEM((CHUNK, H), jnp.float32),
        pltpu.VMEM((CHUNK, H), jnp.float32),
        pltpu.VMEM((NPAD,), jnp.float32),
        pltpu.VMEM_SHARED((NPAD, H), jnp.float32),
        pltpu.SemaphoreType.DMA,
        pltpu.SemaphoreType.DMA,
    ],
    compiler_params=_sc_params,
)(_segsum1_body)


_HI = jax.lax.Precision.HIGHEST


def _layer0_body(x_ref, aa_ref, ab_ref, dp_ref, ws_ref, wn_ref, b_ref,
                 oa_ref, ob_ref, od_ref):
    deg = jnp.sum(dp_ref[0], axis=0)
    dinv = 1.0 / jnp.maximum(deg, 1.0)
    dcol = dinv[:, None]
    a = aa_ref[0] * dcol
    b = ab_ref[0] * dcol
    h = jax.lax.dot(x_ref[...], ws_ref[...], precision=_HI)
    h = h + jax.lax.dot(a, wn_ref[:H, :], precision=_HI)
    h = h + jax.lax.dot(b, wn_ref[H:, :], precision=_HI)
    h = jnp.maximum(h + b_ref[...], 0.0)
    oa_ref[...] = h[:, :H]
    ob_ref[...] = h[:, H:]
    od_ref[...] = dinv.reshape(od_ref.shape)


_R0 = 1024


def _layer0(xp, aggdeg, degp, W0_self, W0_neigh, b0):
    return pl.pallas_call(
        _layer0_body,
        grid=(NPAD // _R0,),
        in_specs=[
            pl.BlockSpec((_R0, D), lambda i: (i, 0)),
            pl.BlockSpec((1, _R0, H), lambda i: (0, i, 0)),
            pl.BlockSpec((1, _R0, H), lambda i: (1, i, 0)),
            pl.BlockSpec((1, 16, _R0), lambda i: (0, 0, i)),
            pl.BlockSpec((D, D), lambda i: (0, 0)),
            pl.BlockSpec((D, D), lambda i: (0, 0)),
            pl.BlockSpec((1, D), lambda i: (0, 0)),
        ],
        out_specs=[
            pl.BlockSpec((_R0, H), lambda i: (i, 0)),
            pl.BlockSpec((_R0, H), lambda i: (i, 0)),
            pl.BlockSpec((_R0 // 128, 128), lambda i: (i, 0)),
        ],
        out_shape=[
            jax.ShapeDtypeStruct((NPAD, H), jnp.float32),
            jax.ShapeDtypeStruct((NPAD, H), jnp.float32),
            jax.ShapeDtypeStruct((NPAD // 128, 128), jnp.float32),
        ],
    )(xp, aggdeg, aggdeg, degp, W0_self, W0_neigh, b0)


def _layer1_body(at_ref, ht_ref, ws_ref, wn_ref, b_ref, o_ref):
    hl = ht_ref[0]
    hh = ht_ref[1]
    o = jax.lax.dot(hl, ws_ref[:H, :], precision=_HI)
    o = o + jax.lax.dot(hh, ws_ref[H:, :], precision=_HI)
    o = o + jax.lax.dot(at_ref[0], wn_ref[:H, :], precision=_HI)
    o = o + jax.lax.dot(at_ref[1], wn_ref[H:, :], precision=_HI)
    o_ref[...] = o + b_ref[...]


def _layer1(aggt, ht, W1_self, W1_neigh, b1):
    return pl.pallas_call(
        _layer1_body,
        out_shape=jax.ShapeDtypeStruct((NT, D), jnp.float32),
    )(aggt, ht, W1_self, W1_neigh, b1)


def kernel(x, edge_index, target_indices, W0_self, W0_neigh, b0,
           W1_self, W1_neigh, b1):
    f32 = jnp.float32
    x = x.astype(f32)
    src = edge_index[0]
    dst = edge_index[1]

    xp = jnp.zeros((NPAD, D), f32).at[:N_NODES].set(x)
    xa = xp[:, :H]
    xb = xp[:, H:]

    pad = EPAD - E
    srcp = jnp.concatenate([src, jnp.zeros((pad,), jnp.int32)]).reshape(
        EPAD // CHUNK, CHUNK)
    dstp = jnp.concatenate([dst, jnp.full((pad,), N_NODES, jnp.int32)]).reshape(
        EPAD // CHUNK, CHUNK)
    zrows = jnp.zeros((ROWS_PER_TILE, H), f32)
    zvec = jnp.zeros((NPAD,), f32)

    agg_raw, degp = _seg0(xa, xb, srcp, dstp, zrows, zvec)
    h1a, h1b, dinv2 = _layer0(xp, agg_raw, degp, W0_self, W0_neigh,
                              b0.reshape(1, D))
    aggt, ht = _seg1(h1a, h1b, srcp, dstp, target_indices,
                     dinv2.reshape(NPAD), zrows)
    out = _layer1(aggt, ht, W1_self, W1_neigh, b1.reshape(1, D))
    return out


# speedup vs baseline: 3.0245x; 1.0242x over previous
import functools

import jax
import jax.numpy as jnp
from jax import lax
from jax.experimental import pallas as pl
from jax.experimental.pallas import tpu as pltpu
from jax.experimental.pallas import tpu_sc as plsc

N_NODES = 10000
NPAD = 10240
D = 256
H = 128
E = 160000
CHUNK = 128
GC = 4
EPAD = 163840
CHUNKS_PER_TILE = EPAD // 16 // CHUNK
GROUPS_PER_TILE = CHUNKS_PER_TILE // GC
EDGES_PER_GROUP = GC * CHUNK
ROWS_PER_TILE = NPAD // 16
NT = 1024
NTT = NT // CHUNK
TPT = CHUNK

_mesh = plsc.VectorSubcoreMesh(core_axis_name="c", subcore_axis_name="s")
_sc_params = pltpu.CompilerParams(needs_layout_passes=False)


def _edge_sweep(x_hbm, src_hbm, dst_hbm, s, srcg, dstg, rows0, rows1, acc,
                gsem0, gsem1, degloc=None):
    ones16 = jnp.ones((16,), jnp.float32)
    rows = (rows0, rows1)
    gsem = (gsem0, gsem1)
    base = s * GROUPS_PER_TILE

    def group(g, carry):
        pltpu.sync_copy(src_hbm.at[pl.ds((base + g) * GC, GC)], srcg)
        pltpu.sync_copy(dst_hbm.at[pl.ds((base + g) * GC, GC)], dstg)
        pltpu.async_copy(x_hbm.at[srcg.at[0]], rows[0], gsem[0])
        for b in range(GC):
            p = b % 2
            if b + 1 < GC:
                pltpu.async_copy(x_hbm.at[srcg.at[b + 1]], rows[1 - p],
                                 gsem[1 - p])
            if degloc is not None:
                for k in range(CHUNK // 16):
                    idx16 = dstg[b, pl.ds(k * 16, 16)]
                    plsc.addupdate_scatter(degloc, [idx16], ones16)
            pltpu.make_async_copy(x_hbm.at[srcg.at[b]], rows[p], gsem[p]).wait()
            pltpu.sync_copy(rows[p], acc.at[dstg.at[b]], add=True)
        return carry

    lax.fori_loop(0, GROUPS_PER_TILE, group, 0)


def _segsum0_body(xa_hbm, xb_hbm, src_hbm, dst_hbm, zrows_hbm, zvec_hbm,
                  agg_hbm, degp_hbm, srcg, dstg, rows0, rows1, degloc, acc,
                  gsem0, gsem1):
    c = lax.axis_index("c")
    s = lax.axis_index("s")
    pltpu.sync_copy(zrows_hbm, acc.at[pl.ds(s * ROWS_PER_TILE, ROWS_PER_TILE)])
    pltpu.sync_copy(zvec_hbm, degloc)
    plsc.subcore_barrier()

    pl.when(c == 0)(lambda: _edge_sweep(
        xa_hbm, src_hbm, dst_hbm, s, srcg, dstg, rows0, rows1, acc,
        gsem0, gsem1, degloc))
    pl.when(c == 1)(lambda: _edge_sweep(
        xb_hbm, src_hbm, dst_hbm, s, srcg, dstg, rows0, rows1, acc,
        gsem0, gsem1, degloc))
    pltpu.sync_copy(degloc, degp_hbm.at[c, s])
    plsc.subcore_barrier()
    pltpu.sync_copy(acc.at[pl.ds(s * ROWS_PER_TILE, ROWS_PER_TILE)],
                    agg_hbm.at[c, pl.ds(s * ROWS_PER_TILE, ROWS_PER_TILE)])


_seg0 = functools.partial(
    pl.kernel,
    mesh=_mesh,
    out_type=[
        jax.ShapeDtypeStruct((2, NPAD, H), jnp.float32),
        jax.ShapeDtypeStruct((2, 16, NPAD), jnp.float32),
    ],
    scratch_types=[
        pltpu.VMEM((GC, CHUNK), jnp.int32),
        pltpu.VMEM((GC, CHUNK), jnp.int32),
        pltpu.VMEM((CHUNK, H), jnp.float32),
        pltpu.VMEM((CHUNK, H), jnp.float32),
        pltpu.VMEM((NPAD,), jnp.float32),
        pltpu.VMEM_SHARED((NPAD, H), jnp.float32),
        pltpu.SemaphoreType.DMA,
        pltpu.SemaphoreType.DMA,
    ],
    compiler_params=_sc_params,
)(_segsum0_body)


def _segsum1_body(ha_hbm, hb_hbm, src_hbm, dst_hbm, ti_hbm, dinv_hbm,
                  zrows_hbm, aggt_hbm, ht_hbm,
                  srcg, dstg, tiv, rows0, rows1, dinvv, acc, gsem0, gsem1):
    c = lax.axis_index("c")
    s = lax.axis_index("s")
    pltpu.sync_copy(zrows_hbm, acc.at[pl.ds(s * ROWS_PER_TILE, ROWS_PER_TILE)])
    pl.when(s < NTT)(
        lambda: pltpu.sync_copy(ti_hbm.at[pl.ds(s * TPT, TPT)], tiv))
    pltpu.sync_copy(dinv_hbm, dinvv)
    plsc.subcore_barrier()

    def run(h_hbm):
        _edge_sweep(h_hbm, src_hbm, dst_hbm, s, srcg, dstg, rows0, rows1,
                    acc, gsem0, gsem1)
        pl.when(s < NTT)(
            lambda: pltpu.async_copy(h_hbm.at[tiv], rows1, gsem1).wait())

    pl.when(c == 0)(lambda: run(ha_hbm))
    pl.when(c == 1)(lambda: run(hb_hbm))
    plsc.subcore_barrier()

    def targets():
        pltpu.async_copy(acc.at[tiv], rows0, gsem0).wait()
        for g in range(TPT // 16):
            tiv16 = tiv[pl.ds(g * 16, 16)]
            dinv16 = plsc.load_gather(dinvv, [tiv16])
            for l in range(16):
                d = dinv16[l]
                r = g * 16 + l
                for k in range(H // 16):
                    rows0[r, pl.ds(k * 16, 16)] = (
                        rows0[r, pl.ds(k * 16, 16)] * d)
        pltpu.sync_copy(rows0, aggt_hbm.at[c, pl.ds(s * TPT, TPT)])
        pltpu.sync_copy(rows1, ht_hbm.at[c, pl.ds(s * TPT, TPT)])

    pl.when(s < NTT)(targets)


_seg1 = functools.partial(
    pl.kernel,
    mesh=_mesh,
    out_type=[
        jax.ShapeDtypeStruct((2, NT, H), jnp.float32),
        jax.ShapeDtypeStruct((2, NT, H), jnp.float32),
    ],
    scratch_types=[
        pltpu.VMEM((GC, CHUNK), jnp.int32),
        pltpu.VMEM((GC, CHUNK), jnp.int32),
        pltpu.VMEM((TPT,), jnp.int32),
        pltpu.VMEM((CHUNK, H), jnp.float32),
        pltpu.VMEM((CHUNK, H), jnp.float32),
        pltpu.VMEM((NPAD,), jnp.float32),
        pltpu.VMEM_SHARED((NPAD, H), jnp.float32),
        pltpu.SemaphoreType.DMA,
        pltpu.SemaphoreType.DMA,
    ],
    compiler_params=_sc_params,
)(_segsum1_body)


_HI = jax.lax.Precision.HIGHEST


def _layer0_body(x_ref, aa_ref, ab_ref, dp_ref, ws_ref, wn_ref, b_ref,
                 oa_ref, ob_ref, od_ref):
    deg = jnp.sum(dp_ref[0], axis=0)
    dinv = 1.0 / jnp.maximum(deg, 1.0)
    dcol = dinv[:, None]
    a = aa_ref[0] * dcol
    b = ab_ref[0] * dcol
    h = jax.lax.dot(x_ref[...], ws_ref[...], precision=_HI)
    h = h + jax.lax.dot(a, wn_ref[:H, :], precision=_HI)
    h = h + jax.lax.dot(b, wn_ref[H:, :], precision=_HI)
    h = jnp.maximum(h + b_ref[...], 0.0)
    oa_ref[...] = h[:, :H]
    ob_ref[...] = h[:, H:]
    od_ref[...] = dinv.reshape(od_ref.shape)


_R0 = 1024


def _layer0(xp, aggdeg, degp, W0_self, W0_neigh, b0):
    return pl.pallas_call(
        _layer0_body,
        grid=(NPAD // _R0,),
        in_specs=[
            pl.BlockSpec((_R0, D), lambda i: (i, 0)),
            pl.BlockSpec((1, _R0, H), lambda i: (0, i, 0)),
            pl.BlockSpec((1, _R0, H), lambda i: (1, i, 0)),
            pl.BlockSpec((1, 16, _R0), lambda i: (0, 0, i)),
            pl.BlockSpec((D, D), lambda i: (0, 0)),
            pl.BlockSpec((D, D), lambda i: (0, 0)),
            pl.BlockSpec((1, D), lambda i: (0, 0)),
        ],
        out_specs=[
            pl.BlockSpec((_R0, H), lambda i: (i, 0)),
            pl.BlockSpec((_R0, H), lambda i: (i, 0)),
            pl.BlockSpec((_R0 // 128, 128), lambda i: (i, 0)),
        ],
        out_shape=[
            jax.ShapeDtypeStruct((NPAD, H), jnp.float32),
            jax.ShapeDtypeStruct((NPAD, H), jnp.float32),
            jax.ShapeDtypeStruct((NPAD // 128, 128), jnp.float32),
        ],
    )(xp, aggdeg, aggdeg, degp, W0_self, W0_neigh, b0)


def _layer1_body(at_ref, ht_ref, ws_ref, wn_ref, b_ref, o_ref):
    hl = ht_ref[0]
    hh = ht_ref[1]
    o = jax.lax.dot(hl, ws_ref[:H, :], precision=_HI)
    o = o + jax.lax.dot(hh, ws_ref[H:, :], precision=_HI)
    o = o + jax.lax.dot(at_ref[0], wn_ref[:H, :], precision=_HI)
    o = o + jax.lax.dot(at_ref[1], wn_ref[H:, :], precision=_HI)
    o_ref[...] = o + b_ref[...]


def _layer1(aggt, ht, W1_self, W1_neigh, b1):
    return pl.pallas_call(
        _layer1_body,
        out_shape=jax.ShapeDtypeStruct((NT, D), jnp.float32),
    )(aggt, ht, W1_self, W1_neigh, b1)


def kernel(x, edge_index, target_indices, W0_self, W0_neigh, b0,
           W1_self, W1_neigh, b1):
    f32 = jnp.float32
    x = x.astype(f32)
    src = edge_index[0]
    dst = edge_index[1]

    xp = jnp.zeros((NPAD, D), f32).at[:N_NODES].set(x)
    xa = xp[:, :H]
    xb = xp[:, H:]

    pad = EPAD - E
    srcp = jnp.concatenate([src, jnp.zeros((pad,), jnp.int32)]).reshape(
        EPAD // CHUNK, CHUNK)
    dstp = jnp.concatenate([dst, jnp.full((pad,), N_NODES, jnp.int32)]).reshape(
        EPAD // CHUNK, CHUNK)
    zrows = jnp.zeros((ROWS_PER_TILE, H), f32)
    zvec = jnp.zeros((NPAD,), f32)

    agg_raw, degp = _seg0(xa, xb, srcp, dstp, zrows, zvec)
    h1a, h1b, dinv2 = _layer0(xp, agg_raw, degp, W0_self, W0_neigh,
                              b0.reshape(1, D))
    aggt, ht = _seg1(h1a, h1b, srcp, dstp, target_indices,
                     dinv2.reshape(NPAD), zrows)
    out = _layer1(aggt, ht, W1_self, W1_neigh, b1.reshape(1, D))
    return out
